# baseline (speedup 1.0000x reference)
import functools

import jax
import jax.numpy as jnp
import numpy as np
from jax import lax
from jax.experimental import pallas as pl
from jax.experimental.pallas import tpu as pltpu
from jax.experimental.pallas import tpu_sc as plsc

_C, _H, _W = 96, 512, 512
_PH, _PW = 16, 16
_NH, _NW = _H // _PH, _W // _PW
_TOTAL = _NH * _NW
_NUM_MIX = 512
_ALPHA = 0.4

_rng = np.random.default_rng(0)
_LAM = float(_rng.beta(_ALPHA, _ALPHA))
_IDX = _rng.choice(_TOTAL, size=_NUM_MIX, replace=False)

_patch_mask = np.zeros(_TOTAL, np.float32)
_patch_mask[_IDX] = 1.0
_W2_ROWS_NP = np.repeat(_patch_mask.reshape(_NH, _NW) * (1.0 - _LAM), _PW, 1).astype(np.float32)
_W2_NP = np.repeat(_W2_ROWS_NP, _PH, 0)
_ACTUAL_LAM = np.float32((_TOTAL - _NUM_MIX + _NUM_MIX * _LAM) / _TOTAL)


_BC = 4


def _blend_body(a_ref, b_ref, w2_ref, o_ref):
    a = a_ref[...]
    o_ref[...] = a + w2_ref[...] * (b_ref[...] - a)


def _kernel_tc(img1, img2):
    w2 = jnp.asarray(_W2_NP)
    out = pl.pallas_call(
        _blend_body,
        out_shape=jax.ShapeDtypeStruct((_C, _H, _W), jnp.float32),
        grid=(_C // _BC,),
        in_specs=[
            pl.BlockSpec((_BC, _H, _W), lambda i: (i, 0, 0)),
            pl.BlockSpec((_BC, _H, _W), lambda i: (i, 0, 0)),
            pl.BlockSpec((_H, _W), lambda i: (0, 0)),
        ],
        out_specs=pl.BlockSpec((_BC, _H, _W), lambda i: (i, 0, 0)),
    )(img1, img2, w2)
    return out, jnp.float32(_ACTUAL_LAM)


_NWORK = 32
_CH_PER_W = _C // _NWORK
_STRIPS = _CH_PER_W * _NH


def _sc_body(w2_hbm, img1_hbm, img2_hbm, out_hbm,
             wv, a0, b0, o0, a1, b1, o1, a2, b2, o2,
             si0, si1, si2, so0, so1, so2):
    wid = lax.axis_index("s") * 2 + lax.axis_index("c")
    ch_base = wid * _CH_PER_W

    def src_idx(t):
        ch = ch_base + t // _NH
        row = (t % _NH) * _PH
        return ch, row

    def in_copies(t, a_ref, b_ref, sem):
        ch, row = src_idx(t)
        return (
            pltpu.make_async_copy(img1_hbm.at[ch, pl.ds(row, _PH), :], a_ref, sem),
            pltpu.make_async_copy(img2_hbm.at[ch, pl.ds(row, _PH), :], b_ref, sem),
        )

    def out_copy(t, o_ref, sem):
        ch, row = src_idx(t)
        return pltpu.make_async_copy(o_ref, out_hbm.at[ch, pl.ds(row, _PH), :], sem)

    def compute(t, a_ref, b_ref, o_ref):
        pr = t % _NH
        for k in range(_NW):
            col = k * 16
            w = wv[pr, pl.ds(col, 16)]
            for r in range(_PH):
                a = a_ref[r, pl.ds(col, 16)]
                b = b_ref[r, pl.ds(col, 16)]
                o_ref[r, pl.ds(col, 16)] = a + w * (b - a)

    pltpu.sync_copy(w2_hbm, wv)

    slots = ((a0, b0, o0, si0, so0), (a1, b1, o1, si1, so1),
             (a2, b2, o2, si2, so2))

    for s in range(3):
        a, b, _, si, _ = slots[s]
        for c in in_copies(s, a, b, si):
            c.start()

    def pair(i, carry):
        for s in range(3):
            a, b, o, si, so = slots[s]
            t = 3 * i + s
            for c in in_copies(t, a, b, si):
                c.wait()

            @pl.when(t >= 3)
            def _wait_prev_out(t=t, o=o, so=so):
                out_copy(t - 3, o, so).wait()

            compute(t, a, b, o)
            out_copy(t, o, so).start()

            @pl.when(t + 3 < _STRIPS)
            def _start_next_in(t=t, a=a, b=b, si=si):
                for c in in_copies(t + 3, a, b, si):
                    c.start()

        return carry

    lax.fori_loop(0, _STRIPS // 3, pair, 0)

    out_copy(_STRIPS - 3, o0, so0).wait()
    out_copy(_STRIPS - 2, o1, so1).wait()
    out_copy(_STRIPS - 1, o2, so2).wait()


def _kernel_sc(img1, img2):
    w2_rows = jnp.asarray(_W2_ROWS_NP)
    run = functools.partial(
        pl.kernel,
        mesh=plsc.VectorSubcoreMesh(core_axis_name="c", subcore_axis_name="s"),
        out_type=jax.ShapeDtypeStruct((_C, _H, _W), jnp.float32),
        scratch_types=[
            pltpu.VMEM((_NH, _W), jnp.float32),
            pltpu.VMEM((_PH, _W), jnp.float32),
            pltpu.VMEM((_PH, _W), jnp.float32),
            pltpu.VMEM((_PH, _W), jnp.float32),
            pltpu.VMEM((_PH, _W), jnp.float32),
            pltpu.VMEM((_PH, _W), jnp.float32),
            pltpu.VMEM((_PH, _W), jnp.float32),
            pltpu.VMEM((_PH, _W), jnp.float32),
            pltpu.VMEM((_PH, _W), jnp.float32),
            pltpu.VMEM((_PH, _W), jnp.float32),
            pltpu.SemaphoreType.DMA,
            pltpu.SemaphoreType.DMA,
            pltpu.SemaphoreType.DMA,
            pltpu.SemaphoreType.DMA,
            pltpu.SemaphoreType.DMA,
            pltpu.SemaphoreType.DMA,
        ],
    )(_sc_body)
    out = run(w2_rows, img1, img2)
    return out, jnp.float32(_ACTUAL_LAM)


kernel = _kernel_sc


# speedup vs baseline: 2.2158x; 2.2158x over previous
import functools

import jax
import jax.numpy as jnp
import numpy as np
from jax import lax
from jax.experimental import pallas as pl
from jax.experimental.pallas import tpu as pltpu
from jax.experimental.pallas import tpu_sc as plsc

_C, _H, _W = 96, 512, 512
_PH, _PW = 16, 16
_NH, _NW = _H // _PH, _W // _PW
_TOTAL = _NH * _NW
_NUM_MIX = 512
_ALPHA = 0.4

_rng = np.random.default_rng(0)
_LAM = float(_rng.beta(_ALPHA, _ALPHA))
_IDX = _rng.choice(_TOTAL, size=_NUM_MIX, replace=False)

_patch_mask = np.zeros(_TOTAL, np.float32)
_patch_mask[_IDX] = 1.0
_W2_ROWS_NP = np.repeat(_patch_mask.reshape(_NH, _NW) * (1.0 - _LAM), _PW, 1).astype(np.float32)
_W2_NP = np.repeat(_W2_ROWS_NP, _PH, 0)
_ACTUAL_LAM = np.float32((_TOTAL - _NUM_MIX + _NUM_MIX * _LAM) / _TOTAL)


_BC = 4


def _blend_body(a_ref, b_ref, w2_ref, o_ref):
    a = a_ref[...]
    o_ref[...] = a + w2_ref[...] * (b_ref[...] - a)


def _kernel_tc(img1, img2):
    w2 = jnp.asarray(_W2_NP)
    out = pl.pallas_call(
        _blend_body,
        out_shape=jax.ShapeDtypeStruct((_C, _H, _W), jnp.float32),
        grid=(_C // _BC,),
        in_specs=[
            pl.BlockSpec((_BC, _H, _W), lambda i: (i, 0, 0)),
            pl.BlockSpec((_BC, _H, _W), lambda i: (i, 0, 0)),
            pl.BlockSpec((_H, _W), lambda i: (0, 0)),
        ],
        out_specs=pl.BlockSpec((_BC, _H, _W), lambda i: (i, 0, 0)),
    )(img1, img2, w2)
    return out, jnp.float32(_ACTUAL_LAM)


_NWORK = 32
_CH_PER_W = _C // _NWORK
_STRIPS = _CH_PER_W * _NH


def _sc_body(w2_hbm, img1_hbm, img2_hbm, out_hbm,
             wv, a0, b0, o0, a1, b1, o1, a2, b2, o2,
             si0, si1, si2, so0, so1, so2):
    wid = lax.axis_index("s") * 2 + lax.axis_index("c")
    ch_base = wid * _CH_PER_W

    def src_idx(t):
        ch = ch_base + t // _NH
        row = (t % _NH) * _PH
        return ch, row

    def in_copies(t, a_ref, b_ref, sem):
        ch, row = src_idx(t)
        return (
            pltpu.make_async_copy(img1_hbm.at[ch, pl.ds(row, _PH), :], a_ref, sem),
            pltpu.make_async_copy(img2_hbm.at[ch, pl.ds(row, _PH), :], b_ref, sem),
        )

    def out_copy(t, o_ref, sem):
        ch, row = src_idx(t)
        return pltpu.make_async_copy(o_ref, out_hbm.at[ch, pl.ds(row, _PH), :], sem)

    def compute(t, a_ref, b_ref, o_ref):
        pr = t % _NH
        for k in range(_NW):
            col = k * 16
            w = wv[pr, pl.ds(col, 16)]
            for r in range(_PH):
                a = a_ref[r, pl.ds(col, 16)]
                b = b_ref[r, pl.ds(col, 16)]
                o_ref[r, pl.ds(col, 16)] = a + w * (b - a)

    pltpu.sync_copy(w2_hbm, wv)

    slots = ((a0, b0, o0, si0, so0), (a1, b1, o1, si1, so1),
             (a2, b2, o2, si2, so2))

    for s in range(3):
        a, b, _, si, _ = slots[s]
        for c in in_copies(s, a, b, si):
            c.start()

    def pair(i, carry):
        for s in range(3):
            a, b, o, si, so = slots[s]
            t = 3 * i + s
            for c in in_copies(t, a, b, si):
                c.wait()

            @pl.when(t >= 3)
            def _wait_prev_out(t=t, o=o, so=so):
                out_copy(t - 3, o, so).wait()

            compute(t, a, b, o)
            out_copy(t, o, so).start()

            @pl.when(t + 3 < _STRIPS)
            def _start_next_in(t=t, a=a, b=b, si=si):
                for c in in_copies(t + 3, a, b, si):
                    c.start()

        return carry

    lax.fori_loop(0, _STRIPS // 3, pair, 0)

    out_copy(_STRIPS - 3, o0, so0).wait()
    out_copy(_STRIPS - 2, o1, so1).wait()
    out_copy(_STRIPS - 1, o2, so2).wait()


def _kernel_sc(img1, img2):
    w2_rows = jnp.asarray(_W2_ROWS_NP)
    run = functools.partial(
        pl.kernel,
        mesh=plsc.VectorSubcoreMesh(core_axis_name="c", subcore_axis_name="s"),
        out_type=jax.ShapeDtypeStruct((_C, _H, _W), jnp.float32),
        scratch_types=[
            pltpu.VMEM((_NH, _W), jnp.float32),
            pltpu.VMEM((_PH, _W), jnp.float32),
            pltpu.VMEM((_PH, _W), jnp.float32),
            pltpu.VMEM((_PH, _W), jnp.float32),
            pltpu.VMEM((_PH, _W), jnp.float32),
            pltpu.VMEM((_PH, _W), jnp.float32),
            pltpu.VMEM((_PH, _W), jnp.float32),
            pltpu.VMEM((_PH, _W), jnp.float32),
            pltpu.VMEM((_PH, _W), jnp.float32),
            pltpu.VMEM((_PH, _W), jnp.float32),
            pltpu.SemaphoreType.DMA,
            pltpu.SemaphoreType.DMA,
            pltpu.SemaphoreType.DMA,
            pltpu.SemaphoreType.DMA,
            pltpu.SemaphoreType.DMA,
            pltpu.SemaphoreType.DMA,
        ],
    )(_sc_body)
    out = run(w2_rows, img1, img2)
    return out, jnp.float32(_ACTUAL_LAM)


kernel = _kernel_tc
